# baseline (speedup 1.0000x reference)
import functools
import math

import jax
import jax.numpy as jnp
from jax import lax
from jax.experimental import pallas as pl
from jax.experimental.pallas import tpu as pltpu
from jax.experimental.pallas import tpu_sc as plsc

_NC = 2
_NS = 16
_L = 16
_NW = _NC * _NS


def _sc_gather(nr_flat, region_emb, r_map):
    M = nr_flat.shape[0]
    D = region_emb.shape[1]
    n_regions = r_map.shape[0]
    per = M // _NW
    mesh = plsc.VectorSubcoreMesh(core_axis_name="c", subcore_axis_name="s")

    @functools.partial(
        pl.kernel,
        out_type=(jax.ShapeDtypeStruct((M, D), jnp.float32),
                  jax.ShapeDtypeStruct((M,), jnp.int32)),
        mesh=mesh,
        scratch_types=[
            pltpu.VMEM((per,), jnp.int32),
            pltpu.VMEM((per, D), jnp.float32),
            pltpu.VMEM((n_regions,), jnp.int32),
            pltpu.VMEM((per,), jnp.int32),
            pltpu.SemaphoreType.DMA,
        ],
        compiler_params=pltpu.CompilerParams(needs_layout_passes=False),
    )
    def sc_body(nr_hbm, re_hbm, rmap_hbm, g_hbm, lr_hbm,
                idx_v, rows_v, rmap_v, lr_v, sem):
        wid = lax.axis_index("s") * _NC + lax.axis_index("c")
        base = wid * per
        pltpu.sync_copy(nr_hbm.at[pl.ds(base, per)], idx_v)
        pltpu.sync_copy(rmap_hbm, rmap_v)
        pltpu.async_copy(re_hbm.at[idx_v], rows_v, sem).wait()
        pltpu.sync_copy(rows_v, g_hbm.at[pl.ds(base, per)])
        for i in range(per // _L):
            idx = idx_v[pl.ds(i * _L, _L)]
            lr_v[pl.ds(i * _L, _L)] = plsc.load_gather(rmap_v, [idx])
        pltpu.sync_copy(lr_v, lr_hbm.at[pl.ds(base, per)])

    return sc_body(nr_flat, region_emb, r_map)


def _decoder_body(ut_ref, g_ref, lr_ref, eids_ref, nsT_ref,
                  eeT_ref, lng_ref, lnb_ref, w1t_ref, b1_ref, w2wt_ref,
                  b2w_ref, w2b_ref, b2b_ref, out_ref):
    f32 = jnp.float32
    B, R, T, Ds = ut_ref.shape
    N = lr_ref.shape[1]
    n_eids = eeT_ref.shape[1]

    iota_r = lax.broadcasted_iota(jnp.int32, (R, N), 0)
    iota_eid = lax.broadcasted_iota(jnp.int32, (n_eids, 1), 0)
    inv_sqrt2 = 1.0 / math.sqrt(2.0)

    for b in range(B):
        eid_oh = (eids_ref[b] == iota_eid).astype(f32)

        gT = jnp.transpose(g_ref[b])
        eT = (nsT_ref[...] + gT
              + jnp.dot(eeT_ref[...], eid_oh, preferred_element_type=f32))

        mu = jnp.mean(eT, axis=0, keepdims=True)
        xc = eT - mu
        var = jnp.mean(xc * xc, axis=0, keepdims=True)
        ehT = xc * lax.rsqrt(var + 1e-5) * lng_ref[...] + lnb_ref[...]

        pre = jnp.dot(w1t_ref[...], ehT, preferred_element_type=f32) + b1_ref[...]
        hT = 0.5 * pre * (1.0 + lax.erf(pre * inv_sqrt2))
        wT = jnp.dot(w2wt_ref[...], hT, preferred_element_type=f32) + b2w_ref[...]
        biasT = jnp.dot(w2b_ref[...], hT, preferred_element_type=f32) + b2b_ref[...]

        lr_row = lr_ref[pl.ds(b, 1), :]
        MT = (lr_row == iota_r).astype(f32)

        u_flat = ut_ref[b].reshape(R * T, Ds)
        pall = jnp.dot(u_flat, wT, preferred_element_type=f32)
        acc = jnp.zeros((T, N), f32)
        for r in range(R):
            acc = acc + pall[r * T:(r + 1) * T, :] * MT[r:r + 1, :]
        out_ref[b] = acc + biasT


def kernel(U, neuron_regions, eids, r_map, neuron_slot, region_emb, eid_emb,
           ln_g, ln_b, W1, b1, W2, b2):
    B, T, R, Ds = U.shape
    N = neuron_regions.shape[1]

    g_flat, lr_flat = _sc_gather(neuron_regions.reshape(B * N), region_emb,
                                 r_map)
    g = g_flat.reshape(B, N, -1)
    lr = lr_flat.reshape(B, N)

    ut = U.transpose(0, 2, 1, 3)
    nsT = neuron_slot[:N].T
    eeT = eid_emb.T
    lng = ln_g.reshape(-1, 1)
    lnb = ln_b.reshape(-1, 1)
    w1t = W1.T
    b1c = b1.reshape(-1, 1)
    w2wt = W2[:, :Ds].T
    b2w = b2[:Ds].reshape(-1, 1)
    w2b = W2[:, Ds].reshape(1, -1)
    b2b = b2[Ds:].reshape(1, 1)

    pred = pl.pallas_call(
        _decoder_body,
        out_shape=jax.ShapeDtypeStruct((B, T, N), jnp.float32),
        in_specs=[
            pl.BlockSpec(memory_space=pltpu.VMEM),
            pl.BlockSpec(memory_space=pltpu.VMEM),
            pl.BlockSpec(memory_space=pltpu.VMEM),
            pl.BlockSpec(memory_space=pltpu.SMEM),
            pl.BlockSpec(memory_space=pltpu.VMEM),
            pl.BlockSpec(memory_space=pltpu.VMEM),
            pl.BlockSpec(memory_space=pltpu.VMEM),
            pl.BlockSpec(memory_space=pltpu.VMEM),
            pl.BlockSpec(memory_space=pltpu.VMEM),
            pl.BlockSpec(memory_space=pltpu.VMEM),
            pl.BlockSpec(memory_space=pltpu.VMEM),
            pl.BlockSpec(memory_space=pltpu.VMEM),
            pl.BlockSpec(memory_space=pltpu.VMEM),
            pl.BlockSpec(memory_space=pltpu.VMEM),
        ],
        out_specs=pl.BlockSpec(memory_space=pltpu.VMEM),
    )(ut, g, lr, eids, nsT, eeT, lng, lnb,
      w1t, b1c, w2wt, b2w, w2b, b2b)
    return pred


# speedup vs baseline: 1.1698x; 1.1698x over previous
import functools
import math

import jax
import jax.numpy as jnp
from jax import lax
from jax.experimental import pallas as pl
from jax.experimental.pallas import tpu as pltpu
from jax.experimental.pallas import tpu_sc as plsc

_NC = 2
_NS = 16
_L = 16
_NW = _NC * _NS


def _sc_gather(nr_flat, region_emb, r_map):
    M = nr_flat.shape[0]
    D = region_emb.shape[1]
    n_regions = r_map.shape[0]
    per = M // _NW
    mesh = plsc.VectorSubcoreMesh(core_axis_name="c", subcore_axis_name="s")

    @functools.partial(
        pl.kernel,
        out_type=(jax.ShapeDtypeStruct((M, D), jnp.float32),
                  jax.ShapeDtypeStruct((M,), jnp.int32)),
        mesh=mesh,
        scratch_types=[
            pltpu.VMEM((per,), jnp.int32),
            pltpu.VMEM((per, D), jnp.float32),
            pltpu.VMEM((n_regions,), jnp.int32),
            pltpu.VMEM((per,), jnp.int32),
            pltpu.SemaphoreType.DMA,
        ],
        compiler_params=pltpu.CompilerParams(needs_layout_passes=False),
    )
    def sc_body(nr_hbm, re_hbm, rmap_hbm, g_hbm, lr_hbm,
                idx_v, rows_v, rmap_v, lr_v, sem):
        wid = lax.axis_index("s") * _NC + lax.axis_index("c")
        base = wid * per
        pltpu.sync_copy(nr_hbm.at[pl.ds(base, per)], idx_v)
        pltpu.sync_copy(rmap_hbm, rmap_v)
        pltpu.async_copy(re_hbm.at[idx_v], rows_v, sem).wait()
        pltpu.sync_copy(rows_v, g_hbm.at[pl.ds(base, per)])
        for i in range(per // _L):
            idx = idx_v[pl.ds(i * _L, _L)]
            lr_v[pl.ds(i * _L, _L)] = plsc.load_gather(rmap_v, [idx])
        pltpu.sync_copy(lr_v, lr_hbm.at[pl.ds(base, per)])

    return sc_body(nr_flat, region_emb, r_map)


def _decoder_body(u_ref, g_ref, lr_ref, eids_ref, ns_ref, ee_ref,
                  lng_ref, lnb_ref, w1_ref, b1_ref, w2_ref, b2_ref, out_ref):
    f32 = jnp.float32
    B, T, R, Ds = u_ref.shape
    N = lr_ref.shape[1]
    n_eids = ee_ref.shape[0]
    d_id = ns_ref.shape[1]

    iota_r = lax.broadcasted_iota(jnp.int32, (R, N), 0)
    iota_eid = lax.broadcasted_iota(jnp.int32, (1, n_eids), 1)
    inv_sqrt2 = 1.0 / math.sqrt(2.0)

    for b in range(B):
        eid_oh = (eids_ref[b] == iota_eid).astype(f32)
        eid_row = jnp.dot(eid_oh, ee_ref[...], preferred_element_type=f32)

        e = ns_ref[...] + g_ref[b] + eid_row

        mu = jnp.mean(e, axis=1, keepdims=True)
        xc = e - mu
        var = jnp.mean(xc * xc, axis=1, keepdims=True)
        eh = xc * lax.rsqrt(var + 1e-5) * lng_ref[...] + lnb_ref[...]

        pre = jnp.dot(eh, w1_ref[...], preferred_element_type=f32) + b1_ref[...]
        h = 0.5 * pre * (1.0 + lax.erf(pre * inv_sqrt2))
        wb = jnp.dot(h, w2_ref[...], preferred_element_type=f32) + b2_ref[...]
        wbT = jnp.transpose(wb)
        wT = wbT[:Ds, :]
        biasT = wbT[Ds:Ds + 1, :]

        lr_row = lr_ref[pl.ds(b, 1), :]
        MT = (lr_row == iota_r).astype(f32)

        u_flat = u_ref[b].reshape(T * R, Ds)
        pall = jnp.dot(u_flat, wT, preferred_element_type=f32)
        pall3 = pall.reshape(T, R, N)
        acc = jnp.sum(pall3 * MT[None, :, :], axis=1)
        out_ref[b] = acc + biasT


def kernel(U, neuron_regions, eids, r_map, neuron_slot, region_emb, eid_emb,
           ln_g, ln_b, W1, b1, W2, b2):
    B, T, R, Ds = U.shape
    N = neuron_regions.shape[1]

    g_flat, lr_flat = _sc_gather(neuron_regions.reshape(B * N), region_emb,
                                 r_map)
    g = g_flat.reshape(B, N, -1)
    lr = lr_flat.reshape(B, N)

    pred = pl.pallas_call(
        _decoder_body,
        out_shape=jax.ShapeDtypeStruct((B, T, N), jnp.float32),
        in_specs=[
            pl.BlockSpec(memory_space=pltpu.VMEM),
            pl.BlockSpec(memory_space=pltpu.VMEM),
            pl.BlockSpec(memory_space=pltpu.VMEM),
            pl.BlockSpec(memory_space=pltpu.SMEM),
            pl.BlockSpec(memory_space=pltpu.VMEM),
            pl.BlockSpec(memory_space=pltpu.VMEM),
            pl.BlockSpec(memory_space=pltpu.VMEM),
            pl.BlockSpec(memory_space=pltpu.VMEM),
            pl.BlockSpec(memory_space=pltpu.VMEM),
            pl.BlockSpec(memory_space=pltpu.VMEM),
            pl.BlockSpec(memory_space=pltpu.VMEM),
            pl.BlockSpec(memory_space=pltpu.VMEM),
        ],
        out_specs=pl.BlockSpec(memory_space=pltpu.VMEM),
    )(U, g, lr, eids, neuron_slot[:N], eid_emb,
      ln_g.reshape(1, -1), ln_b.reshape(1, -1), W1, b1.reshape(1, -1),
      W2, b2.reshape(1, -1))
    return pred
